# Initial kernel scaffold; baseline (speedup 1.0000x reference)
import jax
import jax.numpy as jnp
from jax.experimental import pallas as pl


def kernel(x, W0, W1, W2, W3, W4, W5, W6, W7):
    raise NotImplementedError("write your pallas kernel here")


# speedup vs baseline: 1.3970x; 1.3970x over previous
import functools

import jax
import jax.numpy as jnp
from jax import lax
from jax.experimental import pallas as pl
from jax.experimental.pallas import tpu as pltpu
from jax.experimental.pallas import tpu_sc as plsc

_CYCLES = (7, 30, 91, 365)
_B = 16384
_D = 64
_NC = 2
_NS = 16
_NW = _NC * _NS
_BPW = _B // _NW
_CHUNK = 128
_NCHUNK = _BPW // _CHUNK
_LANES = 16


def _cae_body(x_hbm, t0, t1, t2, t3, t4, t5, t6, t7,
              o0, o1, o2, o3, o4, o5, o6, o7,
              x_v, idx_v, rows_v, sem):
    cid = lax.axis_index("c")
    sid = lax.axis_index("s")
    wid = sid * _NC + cid
    base = wid * _BPW

    pltpu.sync_copy(x_hbm.at[pl.ds(base, _BPW)], x_v)

    def compute_idx(i, carry):
        j = i // (_CHUNK // _LANES)
        k = (i % (_CHUNK // _LANES)) * _LANES
        v = x_v[pl.ds(i * _LANES, _LANES)]
        vf = v.astype(jnp.float32)
        for ci, c in enumerate(_CYCLES):
            q = (vf * jnp.float32(1.0 / c)).astype(jnp.int32)
            r = v - q * c
            r = jnp.where(r < 0, r + c, r)
            r = jnp.where(r >= c, r - c, r)
            idx_v[ci, j, pl.ds(k, _LANES)] = r + 1
        return carry

    lax.fori_loop(0, _BPW // _LANES, compute_idx, 0)

    tables = (t0, t4, t1, t5, t2, t6, t3, t7)
    outs = (o0, o1, o2, o3, o4, o5, o6, o7)
    for k in range(8):
        ci = k // 2
        copies = []
        for j in range(_NCHUNK):
            copies.append(pltpu.async_copy(
                tables[k].at[idx_v.at[ci, j]],
                rows_v.at[pl.ds(j * _CHUNK, _CHUNK)],
                sem))
        for cpy in copies:
            cpy.wait()
        pltpu.sync_copy(rows_v, outs[k].at[pl.ds(base, _BPW)])


@jax.jit
def kernel(x, W0, W1, W2, W3, W4, W5, W6, W7):
    out = jax.ShapeDtypeStruct((_B, _D), jnp.float32)
    mesh = plsc.VectorSubcoreMesh(core_axis_name="c", subcore_axis_name="s")
    run = functools.partial(
        pl.kernel,
        mesh=mesh,
        out_type=[out] * 8,
        compiler_params=pltpu.CompilerParams(use_tc_tiling_on_sc=False),
        scratch_types=[
            pltpu.VMEM((_BPW,), jnp.int32),
            pltpu.VMEM((len(_CYCLES), _NCHUNK, _CHUNK), jnp.int32),
            pltpu.VMEM((_BPW, _D), jnp.float32),
            pltpu.SemaphoreType.DMA,
        ],
    )(_cae_body)
    return tuple(run(x.astype(jnp.int32), W0, W1, W2, W3, W4, W5, W6, W7))


# speedup vs baseline: 1.4118x; 1.0105x over previous
import functools

import jax
import jax.numpy as jnp
from jax import lax
from jax.experimental import pallas as pl
from jax.experimental.pallas import tpu as pltpu
from jax.experimental.pallas import tpu_sc as plsc

_CYCLES = (7, 30, 91, 365)
_B = 16384
_D = 64
_NC = 2
_NS = 16
_NW = _NC * _NS
_BPW = _B // _NW
_CHUNK = 128
_NCHUNK = _BPW // _CHUNK
_LANES = 16


def _cae_body(x_hbm, t0, t1, t2, t3, t4, t5, t6, t7,
              o0, o1, o2, o3, o4, o5, o6, o7,
              x_v, idx_v, rows_a, rows_b, gsem, ssem):
    cid = lax.axis_index("c")
    sid = lax.axis_index("s")
    wid = sid * _NC + cid
    base = wid * _BPW

    pltpu.sync_copy(x_hbm.at[pl.ds(base, _BPW)], x_v)

    def compute_idx(i, carry):
        j = i // (_CHUNK // _LANES)
        k = (i % (_CHUNK // _LANES)) * _LANES
        v = x_v[pl.ds(i * _LANES, _LANES)]
        vf = v.astype(jnp.float32)
        for ci, c in enumerate(_CYCLES):
            q = (vf * jnp.float32(1.0 / c)).astype(jnp.int32)
            r = v - q * c
            r = jnp.where(r < 0, r + c, r)
            r = jnp.where(r >= c, r - c, r)
            idx_v[ci, j, pl.ds(k, _LANES)] = r + 1
        return carry

    lax.fori_loop(0, _BPW // _LANES, compute_idx, 0)

    tables = (t0, t4, t1, t5, t2, t6, t3, t7)
    outs = (o0, o1, o2, o3, o4, o5, o6, o7)
    bufs = (rows_a, rows_b)
    scat = [None, None]
    for k in range(8):
        buf = bufs[k % 2]
        if scat[k % 2] is not None:
            scat[k % 2].wait()
        ci = k // 2
        copies = []
        for j in range(_NCHUNK):
            copies.append(pltpu.async_copy(
                tables[k].at[idx_v.at[ci, j]],
                buf.at[pl.ds(j * _CHUNK, _CHUNK)],
                gsem))
        for cpy in copies:
            cpy.wait()
        scat[k % 2] = pltpu.async_copy(
            buf, outs[k].at[pl.ds(base, _BPW)], ssem)
    scat[0].wait()
    scat[1].wait()


@jax.jit
def kernel(x, W0, W1, W2, W3, W4, W5, W6, W7):
    out = jax.ShapeDtypeStruct((_B, _D), jnp.float32)
    mesh = plsc.VectorSubcoreMesh(core_axis_name="c", subcore_axis_name="s")
    run = functools.partial(
        pl.kernel,
        mesh=mesh,
        out_type=[out] * 8,
        compiler_params=pltpu.CompilerParams(use_tc_tiling_on_sc=False),
        scratch_types=[
            pltpu.VMEM((_BPW,), jnp.int32),
            pltpu.VMEM((len(_CYCLES), _NCHUNK, _CHUNK), jnp.int32),
            pltpu.VMEM((_BPW, _D), jnp.float32),
            pltpu.VMEM((_BPW, _D), jnp.float32),
            pltpu.SemaphoreType.DMA,
            pltpu.SemaphoreType.DMA,
        ],
    )(_cae_body)
    return tuple(run(x.astype(jnp.int32), W0, W1, W2, W3, W4, W5, W6, W7))


# speedup vs baseline: 2.7349x; 1.9373x over previous
import functools

import jax
import jax.numpy as jnp
from jax import lax
from jax.experimental import pallas as pl
from jax.experimental.pallas import tpu as pltpu
from jax.experimental.pallas import tpu_sc as plsc

_CYCLES = (7, 30, 91, 365)
_B = 16384
_D = 64
_NC = 2
_NS = 16
_NW = _NC * _NS
_BPW = _B // _NW
_CHUNK = 128
_NCHUNK = _BPW // _CHUNK
_LANES = 16


def _cae_body(x_hbm, t0, t1, t2, t3, t4, t5, t6, t7,
              o0, o1, o2, o3, o4, o5, o6, o7,
              x_v, idx_v, rows_a, rows_b, gsem, ssem):
    cid = lax.axis_index("c")
    sid = lax.axis_index("s")
    wid = sid * _NC + cid
    base = wid * _BPW

    pltpu.sync_copy(x_hbm.at[pl.ds(base, _BPW)], x_v)

    def compute_idx(i, carry):
        j = i // (_CHUNK // _LANES)
        k = (i % (_CHUNK // _LANES)) * _LANES
        v = x_v[pl.ds(i * _LANES, _LANES)]
        vf = v.astype(jnp.float32)
        for ci, c in enumerate(_CYCLES):
            q = (vf * jnp.float32(1.0 / c)).astype(jnp.int32)
            r = v - q * c
            r = jnp.where(r < 0, r + c, r)
            r = jnp.where(r >= c, r - c, r)
            idx_v[ci, j, pl.ds(k, _LANES)] = r + 1 + wid * (c + 1)
        return carry

    lax.fori_loop(0, _BPW // _LANES, compute_idx, 0)

    tables = (t0, t4, t1, t5, t2, t6, t3, t7)
    outs = (o0, o1, o2, o3, o4, o5, o6, o7)
    bufs = (rows_a, rows_b)
    scat = [None, None]
    for k in range(8):
        buf = bufs[k % 2]
        if scat[k % 2] is not None:
            scat[k % 2].wait()
        ci = k // 2
        copies = []
        for j in range(_NCHUNK):
            copies.append(pltpu.async_copy(
                tables[k].at[idx_v.at[ci, j]],
                buf.at[pl.ds(j * _CHUNK, _CHUNK)],
                gsem))
        for cpy in copies:
            cpy.wait()
        scat[k % 2] = pltpu.async_copy(
            buf, outs[k].at[pl.ds(base, _BPW)], ssem)
    scat[0].wait()
    scat[1].wait()


@jax.jit
def kernel(x, W0, W1, W2, W3, W4, W5, W6, W7):
    out = jax.ShapeDtypeStruct((_B, _D), jnp.float32)
    mesh = plsc.VectorSubcoreMesh(core_axis_name="c", subcore_axis_name="s")
    run = functools.partial(
        pl.kernel,
        mesh=mesh,
        out_type=[out] * 8,
        compiler_params=pltpu.CompilerParams(use_tc_tiling_on_sc=False),
        scratch_types=[
            pltpu.VMEM((_BPW,), jnp.int32),
            pltpu.VMEM((len(_CYCLES), _NCHUNK, _CHUNK), jnp.int32),
            pltpu.VMEM((_BPW, _D), jnp.float32),
            pltpu.VMEM((_BPW, _D), jnp.float32),
            pltpu.SemaphoreType.DMA,
            pltpu.SemaphoreType.DMA,
        ],
    )(_cae_body)
    reps = [jnp.tile(w, (_NW, 1)) for w in (W0, W1, W2, W3, W4, W5, W6, W7)]
    return tuple(run(x.astype(jnp.int32), *reps))


# speedup vs baseline: 3.0668x; 1.1214x over previous
import functools

import jax
import jax.numpy as jnp
from jax import lax
from jax.experimental import pallas as pl
from jax.experimental.pallas import tpu as pltpu
from jax.experimental.pallas import tpu_sc as plsc

_CYCLES = (7, 30, 91, 365)
_B = 16384
_D = 64
_NC = 2
_NS = 16
_NW = _NC * _NS
_BPW = _B // _NW
_CHUNK = 128
_NCHUNK = _BPW // _CHUNK
_LANES = 16


def _cae_body(x_hbm, t0, t1, t2, t3,
              o0, o1, o2, o3,
              x_v, idx_v, rows_a, rows_b, gsem, ssem):
    cid = lax.axis_index("c")
    sid = lax.axis_index("s")
    wid = sid * _NC + cid
    base = wid * _BPW

    pltpu.sync_copy(x_hbm.at[pl.ds(base, _BPW)], x_v)

    def compute_idx(i, carry):
        j = i // (_CHUNK // _LANES)
        k = (i % (_CHUNK // _LANES)) * _LANES
        v = x_v[pl.ds(i * _LANES, _LANES)]
        vf = v.astype(jnp.float32)
        for ci, c in enumerate(_CYCLES):
            q = (vf * jnp.float32(1.0 / c)).astype(jnp.int32)
            r = v - q * c
            r = jnp.where(r < 0, r + c, r)
            r = jnp.where(r >= c, r - c, r)
            idx_v[ci, j, pl.ds(k, _LANES)] = r + 1 + wid * (c + 1)
        return carry

    lax.fori_loop(0, _BPW // _LANES, compute_idx, 0)

    tables = (t0, t1, t2, t3)
    outs = (o0, o1, o2, o3)
    bufs = (rows_a, rows_b)
    scat = [None, None]
    u = 0
    for ci in range(4):
        for j in range(_NCHUNK):
            buf = bufs[u % 2]
            if scat[u % 2] is not None:
                scat[u % 2].wait()
            pltpu.async_copy(
                tables[ci].at[idx_v.at[ci, j]], buf, gsem).wait()
            scat[u % 2] = pltpu.async_copy(
                buf, outs[ci].at[pl.ds(base + j * _CHUNK, _CHUNK)], ssem)
            u += 1
    scat[0].wait()
    scat[1].wait()


@jax.jit
def kernel(x, W0, W1, W2, W3, W4, W5, W6, W7):
    out = jax.ShapeDtypeStruct((_B, 2 * _D), jnp.float32)
    mesh = plsc.VectorSubcoreMesh(core_axis_name="c", subcore_axis_name="s")
    run = functools.partial(
        pl.kernel,
        mesh=mesh,
        out_type=[out] * 4,
        scratch_types=[
            pltpu.VMEM((_BPW,), jnp.int32),
            pltpu.VMEM((len(_CYCLES), _NCHUNK, _CHUNK), jnp.int32),
            pltpu.VMEM((_CHUNK, 2 * _D), jnp.float32),
            pltpu.VMEM((_CHUNK, 2 * _D), jnp.float32),
            pltpu.SemaphoreType.DMA,
            pltpu.SemaphoreType.DMA,
        ],
    )(_cae_body)
    reps = [jnp.tile(jnp.concatenate([ws, wc], axis=1), (_NW, 1))
            for ws, wc in ((W0, W4), (W1, W5), (W2, W6), (W3, W7))]
    p0, p1, p2, p3 = run(x.astype(jnp.int32), *reps)
    return (p0[:, :_D], p0[:, _D:], p1[:, :_D], p1[:, _D:],
            p2[:, :_D], p2[:, _D:], p3[:, :_D], p3[:, _D:])


# speedup vs baseline: 4.5912x; 1.4971x over previous
import functools

import jax
import jax.numpy as jnp
from jax import lax
from jax.experimental import pallas as pl
from jax.experimental.pallas import tpu as pltpu
from jax.experimental.pallas import tpu_sc as plsc

_CYCLES = (7, 30, 91, 365)
_BOUNDS = tuple(c + 1 for c in _CYCLES)
_OFFS = (0, 8, 39, 131)
_ROWS = sum(_BOUNDS)
_B = 16384
_D = 64
_NC = 2
_NS = 16
_NW = _NC * _NS
_BPW = _B // _NW
_BLK = 128
_NBLK = _BPW // _BLK
_LANES = 16
_CHUNK_BYTES = _D * _BLK * 4


def _cae_body(x_hbm, tab_hbm,
              o0, o1, o2, o3, o4, o5, o6, o7,
              x_v, idx_v, tab_v, tb_a, tb_b, ssem):
    cid = lax.axis_index("c")
    sid = lax.axis_index("s")
    wid = sid * _NC + cid
    base = wid * _BPW

    pltpu.sync_copy(tab_hbm, tab_v)
    pltpu.sync_copy(x_hbm.at[pl.ds(base, _BPW)], x_v)

    def compute_idx(i, carry):
        v = x_v[pl.ds(i * _LANES, _LANES)]
        vf = v.astype(jnp.float32)
        for ci, c in enumerate(_CYCLES):
            q = (vf * jnp.float32(1.0 / c)).astype(jnp.int32)
            r = v - q * c
            r = jnp.where(r < 0, r + c, r)
            r = jnp.where(r >= c, r - c, r)
            idx_v[ci, pl.ds(i * _LANES, _LANES)] = r + (1 + _OFFS[ci])
        return carry

    lax.fori_loop(0, _BPW // _LANES, compute_idx, 0)

    iota = lax.iota(jnp.int32, _LANES)
    outs = (o0, o1, o2, o3, o4, o5, o6, o7)
    bufs = (tb_a, tb_b)

    def unit(up, carry):
        for half in range(2):
            u = up * 2 + half
            ci = u // _NBLK
            h = u % _NBLK
            tb = bufs[half]

            @pl.when(u >= 2)
            def _():
                pltpu.make_async_copy(
                    o0.at[pl.ds(0, _D), pl.ds(0, _BLK)],
                    tb.at[pl.ds(0, _D)], ssem).wait()
                pltpu.make_async_copy(
                    o0.at[pl.ds(0, _D), pl.ds(0, _BLK)],
                    tb.at[pl.ds(_D, _D)], ssem).wait()

            def fill(g, c2, tb=tb):
                r16 = idx_v[ci, pl.ds(h * _BLK + g * _LANES, _LANES)]
                b16 = g * _LANES + iota
                for c0 in range(2 * _D):
                    col = (c0 + iota) & (2 * _D - 1)
                    val = plsc.load_gather(tab_v, [r16, col])
                    plsc.store_scatter(tb, [col, b16], val)
                return c2

            lax.fori_loop(0, _BLK // _LANES, fill, 0)

            dst = pl.ds(base + h * _BLK, _BLK)
            for j in range(4):
                @pl.when(ci == j)
                def _(j=j, tb=tb, dst=dst):
                    pltpu.async_copy(
                        tb.at[pl.ds(0, _D)], outs[2 * j].at[:, dst], ssem)
                    pltpu.async_copy(
                        tb.at[pl.ds(_D, _D)], outs[2 * j + 1].at[:, dst],
                        ssem)
        return carry

    lax.fori_loop(0, 8, unit, 0)

    for tb in bufs:
        pltpu.make_async_copy(
            o0.at[pl.ds(0, _D), pl.ds(0, _BLK)],
            tb.at[pl.ds(0, _D)], ssem).wait()
        pltpu.make_async_copy(
            o0.at[pl.ds(0, _D), pl.ds(0, _BLK)],
            tb.at[pl.ds(_D, _D)], ssem).wait()


@jax.jit
def kernel(x, W0, W1, W2, W3, W4, W5, W6, W7):
    mesh = plsc.VectorSubcoreMesh(core_axis_name="c", subcore_axis_name="s")
    run = functools.partial(
        pl.kernel,
        mesh=mesh,
        out_type=[jax.ShapeDtypeStruct((_D, _B), jnp.float32)] * 8,
        compiler_params=pltpu.CompilerParams(needs_layout_passes=False),
        scratch_types=[
            pltpu.VMEM((_BPW,), jnp.int32),
            pltpu.VMEM((len(_CYCLES), _BPW), jnp.int32),
            pltpu.VMEM((_ROWS, 2 * _D), jnp.float32),
            pltpu.VMEM((2 * _D, _BLK), jnp.float32),
            pltpu.VMEM((2 * _D, _BLK), jnp.float32),
            pltpu.SemaphoreType.DMA,
        ],
    )(_cae_body)
    tab = jnp.concatenate(
        [jnp.concatenate([ws, wc], axis=1)
         for ws, wc in ((W0, W4), (W1, W5), (W2, W6), (W3, W7))], axis=0)
    res = run(x.astype(jnp.int32), tab)
    return tuple(r.T for r in res)


# speedup vs baseline: 8.1592x; 1.7771x over previous
import functools

import jax
import jax.numpy as jnp
from jax import lax
from jax.experimental import pallas as pl
from jax.experimental.pallas import tpu as pltpu
from jax.experimental.pallas import tpu_sc as plsc

_CYCLES = (7, 30, 91, 365)
_BOUNDS = tuple(c + 1 for c in _CYCLES)
_OFFS = (0, 8, 39, 131)
_ROWS = sum(_BOUNDS)
_B = 16384
_D = 64
_NC = 2
_NS = 16
_NW = _NC * _NS
_BPW = _B // _NW
_BLK = 128
_NBLK = _BPW // _BLK
_LANES = 16
_CHUNK_BYTES = _D * _BLK * 4


def _cae_body(x_hbm, tab_hbm,
              o0, o1, o2, o3, o4, o5, o6, o7,
              x_v, idx_v, tab_v, tb_a, tb_b, ssem):
    cid = lax.axis_index("c")
    sid = lax.axis_index("s")
    wid = sid * _NC + cid
    base = wid * _BPW

    pltpu.sync_copy(tab_hbm, tab_v)
    pltpu.sync_copy(x_hbm.at[pl.ds(base, _BPW)], x_v)

    def compute_idx(i, carry):
        v = x_v[pl.ds(i * _LANES, _LANES)]
        vf = v.astype(jnp.float32)
        for ci, c in enumerate(_CYCLES):
            q = (vf * jnp.float32(1.0 / c)).astype(jnp.int32)
            r = v - q * c
            r = jnp.where(r < 0, r + c, r)
            r = jnp.where(r >= c, r - c, r)
            idx_v[ci, pl.ds(i * _LANES, _LANES)] = r + (1 + _OFFS[ci])
        return carry

    lax.fori_loop(0, _BPW // _LANES, compute_idx, 0)

    iota = lax.iota(jnp.int32, _LANES)
    outs = (o0, o1, o2, o3, o4, o5, o6, o7)
    bufs = (tb_a, tb_b)

    def unit(up, carry):
        for half in range(2):
            u = up * 2 + half
            ci = u // _NBLK
            h = u % _NBLK
            tb = bufs[half]

            @pl.when(u >= 2)
            def _():
                pltpu.make_async_copy(
                    o0.at[pl.ds(0, _D), pl.ds(0, _BLK)],
                    tb.at[pl.ds(0, _D)], ssem).wait()
                pltpu.make_async_copy(
                    o0.at[pl.ds(0, _D), pl.ds(0, _BLK)],
                    tb.at[pl.ds(_D, _D)], ssem).wait()

            def fill(g, c2, tb=tb):
                r16 = idx_v[ci, pl.ds(h * _BLK + g * _LANES, _LANES)]
                b16 = g * _LANES + iota
                for c0 in range(0, 2 * _D, 8):
                    cols = [(c0 + d + iota) & (2 * _D - 1) for d in range(8)]
                    vals = [plsc.load_gather(tab_v, [r16, col])
                            for col in cols]
                    for col, val in zip(cols, vals):
                        plsc.store_scatter(tb, [col, b16], val)
                return c2

            lax.fori_loop(0, _BLK // _LANES, fill, 0)

            dst = pl.ds(base + h * _BLK, _BLK)
            for j in range(4):
                @pl.when(ci == j)
                def _(j=j, tb=tb, dst=dst):
                    pltpu.async_copy(
                        tb.at[pl.ds(0, _D)], outs[2 * j].at[:, dst], ssem)
                    pltpu.async_copy(
                        tb.at[pl.ds(_D, _D)], outs[2 * j + 1].at[:, dst],
                        ssem)
        return carry

    lax.fori_loop(0, 8, unit, 0)

    for tb in bufs:
        pltpu.make_async_copy(
            o0.at[pl.ds(0, _D), pl.ds(0, _BLK)],
            tb.at[pl.ds(0, _D)], ssem).wait()
        pltpu.make_async_copy(
            o0.at[pl.ds(0, _D), pl.ds(0, _BLK)],
            tb.at[pl.ds(_D, _D)], ssem).wait()


@jax.jit
def kernel(x, W0, W1, W2, W3, W4, W5, W6, W7):
    mesh = plsc.VectorSubcoreMesh(core_axis_name="c", subcore_axis_name="s")
    run = functools.partial(
        pl.kernel,
        mesh=mesh,
        out_type=[jax.ShapeDtypeStruct((_D, _B), jnp.float32)] * 8,
        compiler_params=pltpu.CompilerParams(needs_layout_passes=False),
        scratch_types=[
            pltpu.VMEM((_BPW,), jnp.int32),
            pltpu.VMEM((len(_CYCLES), _BPW), jnp.int32),
            pltpu.VMEM((_ROWS, 2 * _D), jnp.float32),
            pltpu.VMEM((2 * _D, _BLK), jnp.float32),
            pltpu.VMEM((2 * _D, _BLK), jnp.float32),
            pltpu.SemaphoreType.DMA,
        ],
    )(_cae_body)
    tab = jnp.concatenate(
        [jnp.concatenate([ws, wc], axis=1)
         for ws, wc in ((W0, W4), (W1, W5), (W2, W6), (W3, W7))], axis=0)
    res = run(x.astype(jnp.int32), tab)
    return tuple(r.T for r in res)
